# baseline (speedup 1.0000x reference)
import functools

import jax
import jax.numpy as jnp
from jax import lax
from jax.experimental import pallas as pl
from jax.experimental.pallas import tpu as pltpu
from jax.experimental.pallas import tpu_sc as plsc

_L = 16
_IDSR = 50
_LAP = 8
_NREG = 4
_SBL = 16


@functools.cache
def _make_lookup(R, C, D, V):
    info = plsc.get_sparse_core_info()
    NC, NS = info.num_cores, info.num_subcores
    NW = NC * NS
    DE = D // 4
    NSPAN = NW // 4
    rows_per_span = R // NSPAN
    NLAP = rows_per_span // _LAP
    NSB = NLAP // _SBL
    NP = NSB // 2
    FL = _LAP * _IDSR
    NG = FL // _L
    RR = _LAP // _NREG
    mesh = plsc.VectorSubcoreMesh(core_axis_name="c", subcore_axis_name="s")

    @functools.partial(
        pl.kernel,
        mesh=mesh,
        out_type=jax.ShapeDtypeStruct((R, C, D), jnp.float32),
        scratch_types=[
            pltpu.VMEM((V, DE), jnp.float32),
            pltpu.VMEM((_LAP, _IDSR, DE), jnp.float32),
            [pltpu.VMEM((_SBL * FL,), jnp.int32) for _ in range(2)],
            [pltpu.SemaphoreType.DMA for _ in range(_NREG)],
            [pltpu.SemaphoreType.DMA for _ in range(2)],
        ],
    )
    def lookup(table_hbm, idx_hbm, out_hbm, tbl_v, ring, idxb, sreg, sidx):
        wid = lax.axis_index("s") * NC + lax.axis_index("c")
        span = wid // 4
        col = (wid % 4) * DE
        qbase = span * rows_per_span
        fbase = qbase * _IDSR
        pltpu.sync_copy(table_hbm.at[:, pl.ds(col, DE)], tbl_v)

        def idx_load(sb, j):
            pltpu.async_copy(
                idx_hbm.at[pl.ds(fbase + sb * _SBL * FL, _SBL * FL)],
                idxb[j], sidx[j])

        def idx_wait(sb, j):
            pltpu.make_async_copy(
                idx_hbm.at[pl.ds(fbase + sb * _SBL * FL, _SBL * FL)],
                idxb[j], sidx[j]).wait()

        def reg_out(w, r):
            pltpu.async_copy(
                ring.at[pl.ds(r * RR, RR)],
                out_hbm.at[pl.ds(qbase + w * _LAP + r * RR, RR), :,
                           pl.ds(col, DE)],
                sreg[r])

        def reg_wait(w, r):
            pltpu.make_async_copy(
                ring.at[pl.ds(r * RR, RR)],
                out_hbm.at[pl.ds(qbase + w * _LAP + r * RR, RR), :,
                           pl.ds(col, DE)],
                sreg[r]).wait()

        idx_load(0, 0)
        idx_load(1, 1)

        def group(li, j, g):
            vec = idxb[j][pl.ds(li * FL + g * _L, _L)]
            rs = [vec[k] for k in range(_L)]
            if isinstance(g, int):
                qp = [divmod(g * _L + k, _IDSR) for k in range(_L)]
            else:
                qp = []
                for k in range(_L):
                    i = g * _L + k
                    q = i // _IDSR
                    qp.append((q, i - q * _IDSR))

            @plsc.parallel_loop(0, DE // _L, 1, unroll=DE // _L)
            def col_body(jj):
                for k in range(_L):
                    q, p = qp[k]
                    ring[q, p, pl.ds(jj * _L, _L)] = (
                        tbl_v[rs[k], pl.ds(jj * _L, _L)])

        def lap(w, li, j):
            def seg(lo, hi):
                @plsc.parallel_loop(lo, hi, 1, unroll=2)
                def b(g):
                    group(li, j, g)

            def wait_reg(r):
                @pl.when(w >= 1)
                def _():
                    reg_wait(w - 1, r)

            wait_reg(0)
            seg(0, 6)
            wait_reg(1)
            group(li, j, 6)
            reg_out(w, 0)
            seg(7, 12)
            wait_reg(2)
            group(li, j, 12)
            reg_out(w, 1)
            seg(13, 18)
            wait_reg(3)
            group(li, j, 18)
            reg_out(w, 2)
            seg(19, 24)
            group(li, j, 24)
            reg_out(w, 3)

        def body(p, carry):
            for sbl in range(2):
                sb = 2 * p + sbl
                idx_wait(sb, sbl)

                def inner(li, c2):
                    lap(sb * _SBL + li, li, sbl)
                    return c2

                lax.fori_loop(0, _SBL, inner, 0)

                @pl.when(sb < NSB - 2)
                def _(sbl=sbl):
                    idx_load(sb + 2, sbl)
            return carry

        lax.fori_loop(0, NP, body, 0)
        for r in range(_NREG):
            reg_wait(NLAP - 1, r)

    return lookup


def kernel(ids, kernel):
    rows, cols = ids.shape
    idx = ids.reshape(rows * cols).astype(jnp.int32)
    return _make_lookup(rows, cols, kernel.shape[1], kernel.shape[0])(
        kernel, idx)
